# baseline (speedup 1.0000x reference)
import jax
import jax.numpy as jnp
from jax import lax
from jax.experimental import pallas as pl
from jax.experimental.pallas import tpu as pltpu
from jax.experimental.pallas import tpu_sc as plsc

_GAMMA = 12.0
_NC, _NS, _L = 2, 16, 16
_NW = _NC * _NS
_B = 16384
_D = 64
_CHUNK = _B // _NW


def _sc_body(h_hbm, r_hbm, t_hbm, ent_hbm, rel_hbm, out_hbm,
             idx_ht, idx_r, rows_ht, rows_r, out_v, sem0, sem1):
    wid = lax.axis_index("s") * _NC + lax.axis_index("c")
    base = wid * _CHUNK

    pltpu.sync_copy(h_hbm.at[pl.ds(base, _CHUNK)], idx_ht.at[pl.ds(0, _CHUNK)])
    pltpu.sync_copy(t_hbm.at[pl.ds(base, _CHUNK)], idx_ht.at[pl.ds(_CHUNK, _CHUNK)])
    pltpu.sync_copy(r_hbm.at[pl.ds(base, _CHUNK)], idx_r)

    cp0 = pltpu.async_copy(ent_hbm.at[idx_ht], rows_ht, sem0)
    cp1 = pltpu.async_copy(rel_hbm.at[idx_r], rows_r, sem1)
    cp0.wait()
    cp1.wait()

    lanes = lax.iota(jnp.int32, _L)
    last = lanes == (_L - 1)

    def body(i, carry):
        u = jnp.zeros((_L,), jnp.float32)
        for k in range(_D // _L):
            sl = pl.ds(k * _L, _L)
            u += jnp.abs(rows_ht[i, sl] + rows_r[i, sl] - rows_ht[_CHUNK + i, sl])
        c = plsc.cumsum(u)
        plsc.store_compressed(out_v.at[pl.ds(i, _L)], _GAMMA - c, mask=last)
        return carry

    lax.fori_loop(0, _CHUNK, body, 0)

    pltpu.sync_copy(out_v.at[pl.ds(0, _CHUNK)], out_hbm.at[pl.ds(base, _CHUNK)])


def kernel(sample, entity_embedding, relation_embedding):
    mesh = plsc.VectorSubcoreMesh(
        core_axis_name="c", subcore_axis_name="s",
        num_cores=_NC, num_subcores=_NS)
    k = pl.kernel(
        _sc_body,
        out_type=jax.ShapeDtypeStruct((_B,), jnp.float32),
        mesh=mesh,
        compiler_params=pltpu.CompilerParams(
            needs_layout_passes=False, use_tc_tiling_on_sc=False),
        scratch_types=[
            pltpu.VMEM((2 * _CHUNK,), jnp.int32),
            pltpu.VMEM((_CHUNK,), jnp.int32),
            pltpu.VMEM((2 * _CHUNK, _D), jnp.float32),
            pltpu.VMEM((_CHUNK, _D), jnp.float32),
            pltpu.VMEM((_CHUNK + _L,), jnp.float32),
            pltpu.SemaphoreType.DMA,
            pltpu.SemaphoreType.DMA,
        ],
    )
    out = k(sample[:, 0], sample[:, 1], sample[:, 2],
            entity_embedding, relation_embedding)
    return out.reshape(_B, 1)
